# baseline (speedup 1.0000x reference)
import jax
import jax.numpy as jnp
from jax.experimental import pallas as pl
from jax.experimental.pallas import tpu as pltpu

NTOK = 1000000
BN = 32768
BC = 4096
NC = BN // BC
NSTEPS = pl.cdiv(NTOK, BN)
TAIL_START = (NTOK // BC) * BC
TAIL = NTOK - TAIL_START


def _dot(x, w):
    return jax.lax.dot_general(
        x, w, (((1,), (1,)), ((), ())), preferred_element_type=jnp.float32
    )


def _proj_kernel(x_ref, w_hbm, b_ref, out_hbm, wbuf, tbuf, obuf, tobuf, rsems, wsems):
    i = pl.program_id(0)
    x = x_ref[...]

    def issue_reads(step, slot):
        for c in range(NC):
            start = step * BN + c * BC

            @pl.when(start + BC <= NTOK)
            def _():
                pltpu.make_async_copy(
                    w_hbm.at[pl.ds(start, BC), :],
                    wbuf.at[slot, c],
                    rsems.at[slot, c],
                ).start()

        @pl.when(step == NSTEPS - 1)
        def _():
            pltpu.make_async_copy(
                w_hbm.at[pl.ds(TAIL_START, TAIL), :],
                tbuf,
                rsems.at[slot, NC],
            ).start()

    slot = jax.lax.rem(i, 2)

    @pl.when(i == 0)
    def _():
        issue_reads(i, slot)

    issue_reads(i + 1, 1 - slot)

    for c in range(NC):
        start = i * BN + c * BC

        @pl.when(start + BC <= NTOK)
        def _():
            pltpu.make_async_copy(
                w_hbm.at[pl.ds(start, BC), :],
                wbuf.at[slot, c],
                rsems.at[slot, c],
            ).wait()

            @pl.when(i >= 2)
            def _():
                pltpu.make_async_copy(
                    obuf.at[slot, c],
                    out_hbm.at[:, pl.ds((i - 2) * BN + c * BC, BC)],
                    wsems.at[slot, c],
                ).wait()

            lo = c * BC
            obuf[slot, c] = _dot(x, wbuf[slot, c]) + b_ref[:, lo : lo + BC]
            pltpu.make_async_copy(
                obuf.at[slot, c],
                out_hbm.at[:, pl.ds(start, BC)],
                wsems.at[slot, c],
            ).start()

    @pl.when(i == NSTEPS - 1)
    def _():
        lo = TAIL_START - (NSTEPS - 1) * BN
        pltpu.make_async_copy(
            w_hbm.at[pl.ds(TAIL_START, TAIL), :],
            tbuf,
            rsems.at[slot, NC],
        ).wait()
        tobuf[...] = _dot(x, tbuf[...]) + b_ref[:, lo : lo + TAIL]
        pltpu.make_async_copy(
            tobuf,
            out_hbm.at[:, pl.ds(TAIL_START, TAIL)],
            wsems.at[slot, NC],
        ).start()

        for c in range(NC):
            pltpu.make_async_copy(
                obuf.at[1 - slot, c],
                out_hbm.at[:, pl.ds((NSTEPS - 2) * BN + c * BC, BC)],
                wsems.at[1 - slot, c],
            ).wait()
            start = (NSTEPS - 1) * BN + c * BC
            if start + BC <= NTOK:
                pltpu.make_async_copy(
                    obuf.at[slot, c],
                    out_hbm.at[:, pl.ds(start, BC)],
                    wsems.at[slot, c],
                ).wait()
            else:
                pltpu.make_async_copy(
                    obuf.at[slot, c],
                    out_hbm.at[:, pl.ds((NSTEPS - 3) * BN + c * BC, BC)],
                    wsems.at[slot, c],
                ).wait()
        pltpu.make_async_copy(
            tobuf,
            out_hbm.at[:, pl.ds(TAIL_START, TAIL)],
            wsems.at[slot, NC],
        ).wait()


def kernel(inputs, labels, W, b):
    batch, nhid = inputs.shape
    ntokens = W.shape[0]
    b2 = b.reshape(1, ntokens)
    logits = pl.pallas_call(
        _proj_kernel,
        grid=(NSTEPS,),
        in_specs=[
            pl.BlockSpec((batch, nhid), lambda i: (0, 0)),
            pl.BlockSpec(memory_space=pl.ANY),
            pl.BlockSpec((1, BN), lambda i: (0, i)),
        ],
        out_specs=pl.BlockSpec(memory_space=pl.ANY),
        out_shape=jax.ShapeDtypeStruct((batch, ntokens), jnp.float32),
        scratch_shapes=[
            pltpu.VMEM((2, NC, BC, nhid), jnp.float32),
            pltpu.VMEM((TAIL, nhid), jnp.float32),
            pltpu.VMEM((2, NC, batch, BC), jnp.float32),
            pltpu.VMEM((batch, TAIL), jnp.float32),
            pltpu.SemaphoreType.DMA((2, NC + 1)),
            pltpu.SemaphoreType.DMA((2, NC + 1)),
        ],
        compiler_params=pltpu.CompilerParams(
            dimension_semantics=("arbitrary",),
        ),
    )(inputs, W, b2)
    return (logits, labels)


# speedup vs baseline: 1.0006x; 1.0006x over previous
import jax
import jax.numpy as jnp
from jax.experimental import pallas as pl
from jax.experimental.pallas import tpu as pltpu

NTOK = 1000000
BN = 32768
BC = 4096
NC = BN // BC
NSTEPS = pl.cdiv(NTOK, BN)
TAIL_START = (NTOK // BC) * BC
TAIL = NTOK - TAIL_START


def _dot(x, w):
    return jax.lax.dot_general(
        x, w, (((1,), (1,)), ((), ())), preferred_element_type=jnp.float32
    )


def _proj_kernel(x_ref, w_hbm, b_ref, out_hbm, wbuf, tbuf, obuf, tobuf, rsems, wsems):
    i = pl.program_id(0)
    x = x_ref[...]

    def issue_reads(step, slot):
        for c in range(NC):
            start = step * BN + c * BC

            @pl.when(start + BC <= NTOK)
            def _():
                pltpu.make_async_copy(
                    w_hbm.at[pl.ds(start, BC), :],
                    wbuf.at[slot, c],
                    rsems.at[slot, c],
                ).start()

        @pl.when(step == NSTEPS - 1)
        def _():
            pltpu.make_async_copy(
                w_hbm.at[pl.ds(TAIL_START, TAIL), :],
                tbuf,
                rsems.at[slot, NC],
            ).start()

    slot = jax.lax.rem(i, 2)

    @pl.when(i == 0)
    def _():
        issue_reads(i, slot)

    issue_reads(i + 1, 1 - slot)

    for c in range(NC):
        start = i * BN + c * BC

        @pl.when(start + BC <= NTOK)
        def _():
            pltpu.make_async_copy(
                w_hbm.at[pl.ds(start, BC), :],
                wbuf.at[slot, c],
                rsems.at[slot, c],
            ).wait()

            @pl.when(i >= 2)
            def _():
                pltpu.make_async_copy(
                    obuf.at[slot, c],
                    out_hbm.at[:, pl.ds((i - 2) * BN + c * BC, BC)],
                    wsems.at[slot, c],
                ).wait()

            lo = c * BC
            obuf[slot, c] = _dot(x, wbuf[slot, c]) + b_ref[:, lo : lo + BC]
            pltpu.make_async_copy(
                obuf.at[slot, c],
                out_hbm.at[:, pl.ds(start, BC)],
                wsems.at[slot, c],
            ).start(priority=1)

    @pl.when(i == NSTEPS - 1)
    def _():
        lo = TAIL_START - (NSTEPS - 1) * BN
        pltpu.make_async_copy(
            w_hbm.at[pl.ds(TAIL_START, TAIL), :],
            tbuf,
            rsems.at[slot, NC],
        ).wait()
        tobuf[...] = _dot(x, tbuf[...]) + b_ref[:, lo : lo + TAIL]
        pltpu.make_async_copy(
            tobuf,
            out_hbm.at[:, pl.ds(TAIL_START, TAIL)],
            wsems.at[slot, NC],
        ).start(priority=1)

        for c in range(NC):
            pltpu.make_async_copy(
                obuf.at[1 - slot, c],
                out_hbm.at[:, pl.ds((NSTEPS - 2) * BN + c * BC, BC)],
                wsems.at[1 - slot, c],
            ).wait()
            start = (NSTEPS - 1) * BN + c * BC
            if start + BC <= NTOK:
                pltpu.make_async_copy(
                    obuf.at[slot, c],
                    out_hbm.at[:, pl.ds(start, BC)],
                    wsems.at[slot, c],
                ).wait()
            else:
                pltpu.make_async_copy(
                    obuf.at[slot, c],
                    out_hbm.at[:, pl.ds((NSTEPS - 3) * BN + c * BC, BC)],
                    wsems.at[slot, c],
                ).wait()
        pltpu.make_async_copy(
            tobuf,
            out_hbm.at[:, pl.ds(TAIL_START, TAIL)],
            wsems.at[slot, NC],
        ).wait()


def kernel(inputs, labels, W, b):
    batch, nhid = inputs.shape
    ntokens = W.shape[0]
    b2 = b.reshape(1, ntokens)
    logits = pl.pallas_call(
        _proj_kernel,
        grid=(NSTEPS,),
        in_specs=[
            pl.BlockSpec((batch, nhid), lambda i: (0, 0)),
            pl.BlockSpec(memory_space=pl.ANY),
            pl.BlockSpec((1, BN), lambda i: (0, i)),
        ],
        out_specs=pl.BlockSpec(memory_space=pl.ANY),
        out_shape=jax.ShapeDtypeStruct((batch, ntokens), jnp.float32),
        scratch_shapes=[
            pltpu.VMEM((2, NC, BC, nhid), jnp.float32),
            pltpu.VMEM((TAIL, nhid), jnp.float32),
            pltpu.VMEM((2, NC, batch, BC), jnp.float32),
            pltpu.VMEM((batch, TAIL), jnp.float32),
            pltpu.SemaphoreType.DMA((2, NC + 1)),
            pltpu.SemaphoreType.DMA((2, NC + 1)),
        ],
        compiler_params=pltpu.CompilerParams(
            dimension_semantics=("arbitrary",),
        ),
    )(inputs, W, b2)
    return (logits, labels)
